# baseline (speedup 1.0000x reference)
import jax
import jax.numpy as jnp
from jax import lax
from jax.experimental import pallas as pl
from jax.experimental.pallas import tpu as pltpu
from jax.experimental.pallas import tpu_sc as plsc

_NC = 2
_NS = 16
_CHUNK = 128
_BN = 2000


def _dot(a, b):
    return lax.dot_general(a, b, (((1,), (0,)), ((), ())),
                           preferred_element_type=jnp.float32,
                           precision=lax.Precision.DEFAULT)


def _sigmoid(v):
    return 1.0 / (1.0 + jnp.exp(-v))


def _gru_elu(a, x, Wih, Whh, bih, bhh):
    H = x.shape[1]
    gi = _dot(a, Wih) + bih
    gh = _dot(x, Whh) + bhh
    r = _sigmoid(gi[:, :H] + gh[:, :H])
    z = _sigmoid(gi[:, H:2 * H] + gh[:, H:2 * H])
    n = jnp.tanh(gi[:, 2 * H:] + r * gh[:, 2 * H:])
    xn = (1.0 - z) * n + z * x
    return jnp.where(xn > 0, xn, jnp.exp(xn) - 1.0)


def _emb_msg_body(h_ref, We_ref, be_ref, Wm_ref, bm_ref, x_ref, m_ref):
    x = _dot(h_ref[...], We_ref[...]) + be_ref[...]
    x_ref[...] = x
    m_ref[...] = _dot(x, Wm_ref[...]) + bm_ref[...]


def _gru_msg_body(ap_ref, x_ref, Wih_ref, Whh_ref, bih_ref, bhh_ref,
                  Wm_ref, bm_ref, xo_ref, mo_ref):
    a = ap_ref[0] + ap_ref[1]
    xn = _gru_elu(a, x_ref[...], Wih_ref[...], Whh_ref[...],
                  bih_ref[...], bhh_ref[...])
    xo_ref[...] = xn
    mo_ref[...] = _dot(xn, Wm_ref[...]) + bm_ref[...]


def _gru_out_body(ap_ref, x_ref, Wih_ref, Whh_ref, bih_ref, bhh_ref,
                  Wo_ref, bo_ref, o_ref):
    a = ap_ref[0] + ap_ref[1]
    xn = _gru_elu(a, x_ref[...], Wih_ref[...], Whh_ref[...],
                  bih_ref[...], bhh_ref[...])
    logits = _dot(xn, Wo_ref[...]) + bo_ref[...]
    mx = jnp.max(logits, axis=1, keepdims=True)
    sh = logits - mx
    lse = jnp.log(jnp.sum(jnp.exp(sh), axis=1, keepdims=True))
    o_ref[...] = sh - lse


def _segsum_body(m_hbm, z_hbm, src_hbm, dst_hbm, out_hbm,
                 idx_s, idx_d, buf0, buf1, acc, sem0, sem1, sem2, sem3, semz):
    c = lax.axis_index("c")
    s = lax.axis_index("s")
    tile = c * _NS + s
    nchk = src_hbm.shape[0] // (_NC * _NS)
    n_pad = acc.shape[0]

    rows_z = n_pad // _NS
    zero_cp = pltpu.async_copy(z_hbm.at[pl.ds(s * rows_z, rows_z)],
                               acc.at[pl.ds(s * rows_z, rows_z)], semz)

    stg = idx_s.shape[0]
    first = True
    for hh in range(nchk // stg):
        pltpu.sync_copy(src_hbm.at[pl.ds(tile * nchk + hh * stg, stg)], idx_s)
        pltpu.sync_copy(dst_hbm.at[pl.ds(tile * nchk + hh * stg, stg)], idx_d)
        pltpu.async_copy(m_hbm.at[idx_s.at[0]], buf0, sem0)
        pltpu.async_copy(m_hbm.at[idx_s.at[1]], buf1, sem1)
        if first:
            first = False
            zero_cp.wait()
            plsc.subcore_barrier()

        def body(jj, carry):
            j0 = 2 * jj
            j1 = j0 + 1
            pltpu.make_async_copy(m_hbm.at[idx_s.at[j0]], buf0, sem0).wait()
            pltpu.async_copy(buf0, acc.at[idx_d.at[j0]], sem2, add=True)

            pltpu.make_async_copy(m_hbm.at[idx_s.at[j1]], buf1, sem1).wait()
            pltpu.async_copy(buf1, acc.at[idx_d.at[j1]], sem3, add=True)

            pltpu.make_async_copy(buf0, acc.at[idx_d.at[j0]], sem2).wait()
            pltpu.async_copy(m_hbm.at[idx_s.at[j0 + 2]], buf0, sem0)
            pltpu.make_async_copy(buf1, acc.at[idx_d.at[j1]], sem3).wait()
            pltpu.async_copy(m_hbm.at[idx_s.at[j1 + 2]], buf1, sem1)
            return carry

        lax.fori_loop(0, stg // 2 - 1, body, 0)

        pltpu.make_async_copy(m_hbm.at[idx_s.at[stg - 2]], buf0, sem0).wait()
        pltpu.async_copy(buf0, acc.at[idx_d.at[stg - 2]], sem2, add=True)
        pltpu.make_async_copy(m_hbm.at[idx_s.at[stg - 1]], buf1, sem1).wait()
        pltpu.async_copy(buf1, acc.at[idx_d.at[stg - 1]], sem3, add=True)
        pltpu.make_async_copy(buf0, acc.at[idx_d.at[stg - 2]], sem2).wait()
        pltpu.make_async_copy(buf1, acc.at[idx_d.at[stg - 1]], sem3).wait()
    plsc.subcore_barrier()

    pltpu.sync_copy(acc.at[pl.ds(s * rows_z, rows_z)],
                    out_hbm.at[pl.ds(c * n_pad + s * rows_z, rows_z)])


def _segment_sum_sc(m, zeros_pad, srcp, dstp):
    n, h = m.shape
    nchk = srcp.shape[0] // (_NC * _NS)
    n_pad = zeros_pad.shape[0]
    mesh = plsc.VectorSubcoreMesh(core_axis_name="c", subcore_axis_name="s",
                                  num_cores=_NC, num_subcores=_NS)
    f = pl.kernel(
        _segsum_body,
        out_type=jax.ShapeDtypeStruct((_NC * n_pad, h), jnp.float32),
        mesh=mesh,
        scratch_types=[
            pltpu.VMEM((nchk // 2, _CHUNK), jnp.int32),
            pltpu.VMEM((nchk // 2, _CHUNK), jnp.int32),
            pltpu.VMEM((_CHUNK, h), jnp.float32),
            pltpu.VMEM((_CHUNK, h), jnp.float32),
            pltpu.VMEM_SHARED((n_pad, h), jnp.float32),
            pltpu.SemaphoreType.DMA,
            pltpu.SemaphoreType.DMA,
            pltpu.SemaphoreType.DMA,
            pltpu.SemaphoreType.DMA,
            pltpu.SemaphoreType.DMA,
        ],
    )
    return f(m, zeros_pad, srcp, dstp).reshape(_NC, n_pad, h)[:, :n]


def kernel(h, edge_index, etypes, W_emb, b_emb, W_msg0, b_msg0, W_ih0, W_hh0,
           b_ih0, b_hh0, W_msg1, b_msg1, W_ih1, W_hh1, b_ih1, b_hh1,
           W_out, b_out):
    n, f_in = h.shape
    hid = W_emb.shape[1]
    ncls = W_out.shape[1]
    e = edge_index.shape[1]
    tiles = _NC * _NS

    e_per_tile = -(-e // tiles)
    nchk = 8 * (-(-e_per_tile // (8 * _CHUNK)))
    e_pad = nchk * _CHUNK
    src = edge_index[0]
    dst = edge_index[1]
    if e_per_tile * tiles != e:
        pad = e_per_tile * tiles - e
        src = jnp.concatenate([src, jnp.zeros((pad,), jnp.int32)])
        dst = jnp.concatenate([dst, jnp.full((pad,), n, jnp.int32)])
    srcp = jnp.pad(src.reshape(tiles, e_per_tile),
                   ((0, 0), (0, e_pad - e_per_tile))).reshape(tiles * nchk, _CHUNK)
    dstp = jnp.pad(dst.reshape(tiles, e_per_tile),
                   ((0, 0), (0, e_pad - e_per_tile)),
                   constant_values=n).reshape(tiles * nchk, _CHUNK)

    n_pad = 8 * _NS * (-(-(n + 1) // (8 * _NS)))
    zeros_pad = jnp.zeros((n_pad, hid), jnp.float32)

    grid = (n // _BN,)
    row_blk = pl.BlockSpec((_BN, hid), lambda i: (i, 0))
    ap_blk = pl.BlockSpec((_NC, _BN, hid), lambda i: (0, i, 0))

    def full(shape):
        return pl.BlockSpec(shape, lambda i: tuple(0 for _ in shape))

    b_emb2 = b_emb.reshape(1, hid)
    b_msg0_2 = b_msg0.reshape(1, hid)
    b_msg1_2 = b_msg1.reshape(1, hid)
    b_ih0_2 = b_ih0.reshape(1, 3 * hid)
    b_hh0_2 = b_hh0.reshape(1, 3 * hid)
    b_ih1_2 = b_ih1.reshape(1, 3 * hid)
    b_hh1_2 = b_hh1.reshape(1, 3 * hid)
    b_out2 = b_out.reshape(1, ncls)

    x0, m0 = pl.pallas_call(
        _emb_msg_body,
        grid=grid,
        in_specs=[
            pl.BlockSpec((_BN, f_in), lambda i: (i, 0)),
            full((f_in, hid)), full((1, hid)),
            full((hid, hid)), full((1, hid)),
        ],
        out_specs=[row_blk, row_blk],
        out_shape=[jax.ShapeDtypeStruct((n, hid), jnp.float32),
                   jax.ShapeDtypeStruct((n, hid), jnp.float32)],
    )(h, W_emb, b_emb2, W_msg0, b_msg0_2)

    ap0 = _segment_sum_sc(m0, zeros_pad, srcp, dstp)

    x1, m1 = pl.pallas_call(
        _gru_msg_body,
        grid=grid,
        in_specs=[
            ap_blk, row_blk,
            full((hid, 3 * hid)), full((hid, 3 * hid)),
            full((1, 3 * hid)), full((1, 3 * hid)),
            full((hid, hid)), full((1, hid)),
        ],
        out_specs=[row_blk, row_blk],
        out_shape=[jax.ShapeDtypeStruct((n, hid), jnp.float32),
                   jax.ShapeDtypeStruct((n, hid), jnp.float32)],
    )(ap0, x0, W_ih0, W_hh0, b_ih0_2, b_hh0_2, W_msg1, b_msg1_2)

    ap1 = _segment_sum_sc(m1, zeros_pad, srcp, dstp)

    out = pl.pallas_call(
        _gru_out_body,
        grid=grid,
        in_specs=[
            ap_blk, row_blk,
            full((hid, 3 * hid)), full((hid, 3 * hid)),
            full((1, 3 * hid)), full((1, 3 * hid)),
            full((hid, ncls)), full((1, ncls)),
        ],
        out_specs=pl.BlockSpec((_BN, ncls), lambda i: (i, 0)),
        out_shape=jax.ShapeDtypeStruct((n, ncls), jnp.float32),
    )(ap1, x1, W_ih1, W_hh1, b_ih1_2, b_hh1_2, W_out, b_out2)

    return out


# speedup vs baseline: 1.0775x; 1.0775x over previous
import jax
import jax.numpy as jnp
from jax import lax
from jax.experimental import pallas as pl
from jax.experimental.pallas import tpu as pltpu
from jax.experimental.pallas import tpu_sc as plsc

_NC = 2
_NS = 16
_CHUNK = 128
_BN = 2000


def _dot(a, b):
    return lax.dot_general(a, b, (((1,), (0,)), ((), ())),
                           preferred_element_type=jnp.float32,
                           precision=lax.Precision.DEFAULT)


def _sigmoid(v):
    return 1.0 / (1.0 + jnp.exp(-v))


def _gru_elu(a, x, Wih, Whh, bih, bhh):
    H = x.shape[1]
    gi = _dot(a, Wih) + bih
    gh = _dot(x, Whh) + bhh
    r = _sigmoid(gi[:, :H] + gh[:, :H])
    z = _sigmoid(gi[:, H:2 * H] + gh[:, H:2 * H])
    n = jnp.tanh(gi[:, 2 * H:] + r * gh[:, 2 * H:])
    xn = (1.0 - z) * n + z * x
    return jnp.where(xn > 0, xn, jnp.exp(xn) - 1.0)


def _emb_msg_body(h_ref, We_ref, be_ref, Wm_ref, bm_ref, x_ref, m_ref):
    x = _dot(h_ref[...], We_ref[...]) + be_ref[...]
    x_ref[...] = x
    m_ref[...] = _dot(x, Wm_ref[...]) + bm_ref[...]


def _gru_msg_body(ap_ref, x_ref, Wih_ref, Whh_ref, bih_ref, bhh_ref,
                  Wm_ref, bm_ref, xo_ref, mo_ref):
    a = ap_ref[0] + ap_ref[1]
    xn = _gru_elu(a, x_ref[...], Wih_ref[...], Whh_ref[...],
                  bih_ref[...], bhh_ref[...])
    xo_ref[...] = xn
    mo_ref[...] = _dot(xn, Wm_ref[...]) + bm_ref[...]


def _gru_out_body(ap_ref, x_ref, Wih_ref, Whh_ref, bih_ref, bhh_ref,
                  Wo_ref, bo_ref, o_ref):
    a = ap_ref[0] + ap_ref[1]
    xn = _gru_elu(a, x_ref[...], Wih_ref[...], Whh_ref[...],
                  bih_ref[...], bhh_ref[...])
    logits = _dot(xn, Wo_ref[...]) + bo_ref[...]
    mx = jnp.max(logits, axis=1, keepdims=True)
    sh = logits - mx
    lse = jnp.log(jnp.sum(jnp.exp(sh), axis=1, keepdims=True))
    o_ref[...] = sh - lse


def _segsum_body(m_hbm, z_hbm, src_hbm, dst_hbm, out_hbm,
                 idx_s, idx_d, buf0, buf1, acc, sem0, sem1, sem2, sem3, semz):
    c = lax.axis_index("c")
    s = lax.axis_index("s")
    tile = c * _NS + s
    nchk = src_hbm.shape[0] // (_NC * _NS)
    n_pad = acc.shape[0]

    rows_z = n_pad // _NS
    zero_cp = pltpu.async_copy(z_hbm.at[pl.ds(s * rows_z, rows_z)],
                               acc.at[pl.ds(s * rows_z, rows_z)], semz)

    stg = idx_s.shape[0]
    first = True
    for hh in range(nchk // stg):
        pltpu.sync_copy(src_hbm.at[pl.ds(tile * nchk + hh * stg, stg)], idx_s)
        pltpu.sync_copy(dst_hbm.at[pl.ds(tile * nchk + hh * stg, stg)], idx_d)
        pltpu.async_copy(m_hbm.at[idx_s.at[0]], buf0, sem0)
        pltpu.async_copy(m_hbm.at[idx_s.at[1]], buf1, sem1)
        if first:
            first = False
            zero_cp.wait()
            plsc.subcore_barrier()

        def body(jj, carry):
            j0 = 2 * jj
            j1 = j0 + 1
            pltpu.make_async_copy(m_hbm.at[idx_s.at[j0]], buf0, sem0).wait()
            pltpu.sync_copy(buf0, acc.at[idx_d.at[j0]], add=True)
            pltpu.async_copy(m_hbm.at[idx_s.at[j0 + 2]], buf0, sem0)

            pltpu.make_async_copy(m_hbm.at[idx_s.at[j1]], buf1, sem1).wait()
            pltpu.sync_copy(buf1, acc.at[idx_d.at[j1]], add=True)
            pltpu.async_copy(m_hbm.at[idx_s.at[j1 + 2]], buf1, sem1)
            return carry

        lax.fori_loop(0, stg // 2 - 1, body, 0)

        pltpu.make_async_copy(m_hbm.at[idx_s.at[stg - 2]], buf0, sem0).wait()
        pltpu.sync_copy(buf0, acc.at[idx_d.at[stg - 2]], add=True)
        pltpu.make_async_copy(m_hbm.at[idx_s.at[stg - 1]], buf1, sem1).wait()
        pltpu.sync_copy(buf1, acc.at[idx_d.at[stg - 1]], add=True)
    plsc.subcore_barrier()

    pltpu.sync_copy(acc.at[pl.ds(s * rows_z, rows_z)],
                    out_hbm.at[pl.ds(c * n_pad + s * rows_z, rows_z)])


def _segment_sum_sc(m, zeros_pad, srcp, dstp):
    n, h = m.shape
    nchk = srcp.shape[0] // (_NC * _NS)
    n_pad = zeros_pad.shape[0]
    mesh = plsc.VectorSubcoreMesh(core_axis_name="c", subcore_axis_name="s",
                                  num_cores=_NC, num_subcores=_NS)
    f = pl.kernel(
        _segsum_body,
        out_type=jax.ShapeDtypeStruct((_NC * n_pad, h), jnp.float32),
        mesh=mesh,
        scratch_types=[
            pltpu.VMEM((nchk // 2, _CHUNK), jnp.int32),
            pltpu.VMEM((nchk // 2, _CHUNK), jnp.int32),
            pltpu.VMEM((_CHUNK, h), jnp.float32),
            pltpu.VMEM((_CHUNK, h), jnp.float32),
            pltpu.VMEM_SHARED((n_pad, h), jnp.float32),
            pltpu.SemaphoreType.DMA,
            pltpu.SemaphoreType.DMA,
            pltpu.SemaphoreType.DMA,
            pltpu.SemaphoreType.DMA,
            pltpu.SemaphoreType.DMA,
        ],
    )
    return f(m, zeros_pad, srcp, dstp).reshape(_NC, n_pad, h)[:, :n]


def kernel(h, edge_index, etypes, W_emb, b_emb, W_msg0, b_msg0, W_ih0, W_hh0,
           b_ih0, b_hh0, W_msg1, b_msg1, W_ih1, W_hh1, b_ih1, b_hh1,
           W_out, b_out):
    n, f_in = h.shape
    hid = W_emb.shape[1]
    ncls = W_out.shape[1]
    e = edge_index.shape[1]
    tiles = _NC * _NS

    e_per_tile = -(-e // tiles)
    nchk = 8 * (-(-e_per_tile // (8 * _CHUNK)))
    e_pad = nchk * _CHUNK
    src = edge_index[0]
    dst = edge_index[1]
    if e_per_tile * tiles != e:
        pad = e_per_tile * tiles - e
        src = jnp.concatenate([src, jnp.zeros((pad,), jnp.int32)])
        dst = jnp.concatenate([dst, jnp.full((pad,), n, jnp.int32)])
    srcp = jnp.pad(src.reshape(tiles, e_per_tile),
                   ((0, 0), (0, e_pad - e_per_tile))).reshape(tiles * nchk, _CHUNK)
    dstp = jnp.pad(dst.reshape(tiles, e_per_tile),
                   ((0, 0), (0, e_pad - e_per_tile)),
                   constant_values=n).reshape(tiles * nchk, _CHUNK)

    n_pad = 8 * _NS * (-(-(n + 1) // (8 * _NS)))
    zeros_pad = jnp.zeros((n_pad, hid), jnp.float32)

    grid = (n // _BN,)
    row_blk = pl.BlockSpec((_BN, hid), lambda i: (i, 0))
    ap_blk = pl.BlockSpec((_NC, _BN, hid), lambda i: (0, i, 0))

    def full(shape):
        return pl.BlockSpec(shape, lambda i: tuple(0 for _ in shape))

    b_emb2 = b_emb.reshape(1, hid)
    b_msg0_2 = b_msg0.reshape(1, hid)
    b_msg1_2 = b_msg1.reshape(1, hid)
    b_ih0_2 = b_ih0.reshape(1, 3 * hid)
    b_hh0_2 = b_hh0.reshape(1, 3 * hid)
    b_ih1_2 = b_ih1.reshape(1, 3 * hid)
    b_hh1_2 = b_hh1.reshape(1, 3 * hid)
    b_out2 = b_out.reshape(1, ncls)

    x0, m0 = pl.pallas_call(
        _emb_msg_body,
        grid=grid,
        in_specs=[
            pl.BlockSpec((_BN, f_in), lambda i: (i, 0)),
            full((f_in, hid)), full((1, hid)),
            full((hid, hid)), full((1, hid)),
        ],
        out_specs=[row_blk, row_blk],
        out_shape=[jax.ShapeDtypeStruct((n, hid), jnp.float32),
                   jax.ShapeDtypeStruct((n, hid), jnp.float32)],
    )(h, W_emb, b_emb2, W_msg0, b_msg0_2)

    ap0 = _segment_sum_sc(m0, zeros_pad, srcp, dstp)

    x1, m1 = pl.pallas_call(
        _gru_msg_body,
        grid=grid,
        in_specs=[
            ap_blk, row_blk,
            full((hid, 3 * hid)), full((hid, 3 * hid)),
            full((1, 3 * hid)), full((1, 3 * hid)),
            full((hid, hid)), full((1, hid)),
        ],
        out_specs=[row_blk, row_blk],
        out_shape=[jax.ShapeDtypeStruct((n, hid), jnp.float32),
                   jax.ShapeDtypeStruct((n, hid), jnp.float32)],
    )(ap0, x0, W_ih0, W_hh0, b_ih0_2, b_hh0_2, W_msg1, b_msg1_2)

    ap1 = _segment_sum_sc(m1, zeros_pad, srcp, dstp)

    out = pl.pallas_call(
        _gru_out_body,
        grid=grid,
        in_specs=[
            ap_blk, row_blk,
            full((hid, 3 * hid)), full((hid, 3 * hid)),
            full((1, 3 * hid)), full((1, 3 * hid)),
            full((hid, ncls)), full((1, ncls)),
        ],
        out_specs=pl.BlockSpec((_BN, ncls), lambda i: (i, 0)),
        out_shape=jax.ShapeDtypeStruct((n, ncls), jnp.float32),
    )(ap1, x1, W_ih1, W_hh1, b_ih1_2, b_hh1_2, W_out, b_out2)

    return out
